# baseline (speedup 1.0000x reference)
import jax
import jax.numpy as jnp
from jax import lax
from jax.experimental import pallas as pl
from jax.experimental.pallas import tpu as pltpu
from jax.experimental.pallas import tpu_sc as plsc

N_IN = 2048
N_OUT = 2048
HALF_ROWS = 32
HALF_W = HALF_ROWS * N_IN
SUB = 8192
STRIDE = 512


def _sc_densify_body(nnz, bounds_hbm, flat_hbm, vals_hbm, tflat_hbm, tval_hbm,
                     w_hbm, bnd_v, idx_v, val_v, tidx_v, tvl_v, wbuf):
    c = lax.axis_index("c")
    s = lax.axis_index("s")
    wid = s * 2 + c
    pltpu.sync_copy(bounds_hbm, bnd_v)
    pltpu.sync_copy(tflat_hbm, tidx_v)
    pltpu.sync_copy(tval_hbm, tvl_v)
    zero16 = jnp.zeros((16,), jnp.float32)
    lanes0 = lax.iota(jnp.int32, 16)
    last8 = (nnz - SUB) & ~7

    for h in range(2):
        blk = wid * 2 + h
        bvec = bnd_v[pl.ds(blk, 16)]
        b0 = bvec[0]
        b1 = bvec[1]
        base = blk * HALF_W

        def zbody(i, _):
            r = i // (N_IN // 16)
            col = (i % (N_IN // 16)) * 16
            wbuf[r, pl.ds(col, 16)] = zero16
            return 0
        lax.fori_loop(0, HALF_W // 16, zbody, 0, unroll=8)

        s0 = b0 & ~7
        nsub = (b1 - s0 + SUB - 1) // SUB

        def sub_body(j, _):
            start = jnp.minimum(s0 + j * SUB, last8)
            start = pl.multiple_of(start, 8)
            pltpu.sync_copy(flat_hbm.at[pl.ds(start, SUB)], idx_v)
            pltpu.sync_copy(vals_hbm.at[pl.ds(start, SUB)], val_v)
            lo = b0 - start
            hi = b1 - start

            def scat(i, _):
                lane = lanes0 + i * 16
                m = (lane >= lo) & (lane < hi)
                fi = idx_v[pl.ds(i * 16, 16)] - base
                v = val_v[pl.ds(i * 16, 16)]
                plsc.store_scatter(
                    wbuf,
                    [lax.shift_right_logical(fi, 11), fi & (N_IN - 1)],
                    v, mask=m)
                return 0
            lax.fori_loop(0, SUB // 16, scat, 0, unroll=8)
            return 0
        lax.fori_loop(0, nsub, sub_body, 0)

        tfi = tidx_v[...] - base
        tmask = (tfi >= 0) & (tfi < HALF_W)
        plsc.store_scatter(
            wbuf,
            [lax.shift_right_logical(tfi, 11), tfi & (N_IN - 1)],
            tvl_v[...], mask=tmask)

        pltpu.sync_copy(wbuf, w_hbm.at[pl.ds(blk * HALF_ROWS, HALF_ROWS), :])


def _densify(bounds, flat, vals):
    mesh = plsc.VectorSubcoreMesh(core_axis_name="c", subcore_axis_name="s")
    nnz = flat.shape[0]
    body = lambda *refs: _sc_densify_body(nnz, *refs)
    return pl.kernel(
        body,
        out_type=jax.ShapeDtypeStruct((N_OUT, N_IN), jnp.float32),
        mesh=mesh,
        scratch_types=[
            pltpu.VMEM((128,), jnp.int32),
            pltpu.VMEM((SUB,), jnp.int32),
            pltpu.VMEM((SUB,), jnp.float32),
            pltpu.VMEM((16,), jnp.int32),
            pltpu.VMEM((16,), jnp.float32),
            pltpu.VMEM((HALF_ROWS, N_IN), jnp.float32),
        ],
        compiler_params=pltpu.CompilerParams(needs_layout_passes=False),
    )(bounds, flat, vals, flat[nnz - 16:], vals[nnz - 16:])


BM = 512


def _mm_body(x_ref, w_ref, b_ref, o_ref):
    xb = x_ref[...].astype(jnp.bfloat16)
    wb = w_ref[...].astype(jnp.bfloat16)
    acc = lax.dot_general(xb, wb, (((1,), (1,)), ((), ())),
                          preferred_element_type=jnp.float32)
    o_ref[...] = acc + b_ref[...]


def _matmul(x, w, bias2):
    batch = x.shape[0]
    return pl.pallas_call(
        _mm_body,
        grid=(batch // BM,),
        in_specs=[
            pl.BlockSpec((BM, N_IN), lambda i: (i, 0)),
            pl.BlockSpec((N_OUT, N_IN), lambda i: (0, 0)),
            pl.BlockSpec((1, N_OUT), lambda i: (0, 0)),
        ],
        out_specs=pl.BlockSpec((BM, N_OUT), lambda i: (i, 0)),
        out_shape=jax.ShapeDtypeStruct((batch, N_OUT), jnp.float32),
    )(x, w, bias2)


def _chunk_bounds(rows):
    nnz = rows.shape[0]
    q = jnp.arange(0, N_OUT + 1, HALF_ROWS, dtype=rows.dtype)
    r_sub = rows[::STRIDE]
    jc = jnp.sum(r_sub[None, :] < q[:, None], axis=1)
    starts = jnp.clip((jc - 1) * STRIDE, 0, nnz - STRIDE)
    win = jax.vmap(lambda st: lax.dynamic_slice(rows, (st,), (STRIDE,)))(starts)
    return (starts + jnp.sum(win < q[:, None], axis=1)).astype(jnp.int32)


def kernel(in_values, values, indices, bias):
    rows = indices[0].astype(jnp.int32)
    cols = indices[1].astype(jnp.int32)
    flat = rows * N_IN + cols
    bounds = jnp.pad(_chunk_bounds(rows), (0, 63))
    w = _densify(bounds, flat, values)
    out = _matmul(in_values, w, bias.reshape(1, N_OUT))
    return out


# speedup vs baseline: 1.9160x; 1.9160x over previous
import jax
import jax.numpy as jnp
from jax import lax
from jax.experimental import pallas as pl
from jax.experimental.pallas import tpu as pltpu
from jax.experimental.pallas import tpu_sc as plsc

N_IN = 2048
N_OUT = 2048
LOG_IN = 11
HALF_ROWS = 32
HALF_W = HALF_ROWS * N_IN
SUB = 8192
STRIDE = 512


def _sc_densify_body(nnz, coarse_hbm, flat_hbm, vals_hbm,
                     tflat_hbm, tval_hbm, w_hbm,
                     bnd_v, win_v, iA, vA, iB, vB,
                     ti_v, tv_v, wbuf, semA, semB):
    c = lax.axis_index("c")
    s = lax.axis_index("s")
    t2 = (s * 2 + c) * 2
    pltpu.sync_copy(coarse_hbm, bnd_v)
    pltpu.sync_copy(tflat_hbm, ti_v)
    pltpu.sync_copy(tval_hbm, tv_v)
    zero16 = jnp.zeros((16,), jnp.float32)
    lanes0 = lax.iota(jnp.int32, 16)
    last8 = (nnz - SUB) & ~7
    wsmax = (nnz - STRIDE) & ~7
    tflat = ti_v[...]
    tpos = lanes0 + (nnz - 16)

    jcv = bnd_v[pl.ds(t2, 16)]
    ws_list, handles = [], []
    for k in range(3):
        ws = jnp.minimum(jnp.maximum((jcv[k] - 1) * STRIDE, 0), wsmax)
        ws = pl.multiple_of(ws, 8)
        ws_list.append(ws)
        handles.append(pltpu.async_copy(
            flat_hbm.at[pl.ds(ws, STRIDE)],
            win_v.at[pl.ds(k * STRIDE, STRIDE)], semA))
    for hd in handles:
        hd.wait()
    bnds = []
    for k in range(3):
        qf = (t2 + k) * HALF_W
        def cbody(i, acc, k=k, qf=qf):
            v = win_v[pl.ds(k * STRIDE + i * 16, 16)]
            return acc + plsc.all_reduce_population_count(v < qf)
        acc = lax.fori_loop(0, STRIDE // 16, cbody,
                            jnp.zeros((16,), jnp.int32), unroll=8)
        tf = plsc.all_reduce_population_count(
            (tpos >= ws_list[k] + STRIDE) & (tflat < qf))
        bnds.append(ws_list[k] + acc[0] + tf[0])

    def issue(start, i_b, v_b, sem):
        pltpu.async_copy(flat_hbm.at[pl.ds(start, SUB)], i_b, sem)
        pltpu.async_copy(vals_hbm.at[pl.ds(start, SUB)], v_b, sem)

    def drain(i_b, v_b, sem):
        pltpu.make_async_copy(flat_hbm.at[pl.ds(0, SUB)], i_b, sem).wait()
        pltpu.make_async_copy(vals_hbm.at[pl.ds(0, SUB)], v_b, sem).wait()

    for h in range(2):
        blk = t2 + h
        b0, b1 = bnds[h], bnds[h + 1]
        base = blk * HALF_W
        s0 = b0 & ~7

        def sub_start(j):
            st = jnp.minimum(s0 + j * SUB, last8)
            return pl.multiple_of(st, 8)

        issue(sub_start(0), iA, vA, semA)

        def zbody(i, _):
            r = i // (N_IN // 16)
            col = (i % (N_IN // 16)) * 16
            wbuf[r, pl.ds(col, 16)] = zero16
            return 0
        lax.fori_loop(0, HALF_W // 16, zbody, 0, unroll=8)

        def scatter(j, i_b, v_b):
            start = sub_start(j)
            lo = b0 - start
            hi = b1 - start

            def scat(i, _):
                lane = lanes0 + i * 16
                m = (lane >= lo) & (lane < hi)
                fi = i_b[pl.ds(i * 16, 16)] - base
                v = v_b[pl.ds(i * 16, 16)]
                plsc.store_scatter(
                    wbuf,
                    [lax.shift_right_logical(fi, LOG_IN), fi & (N_IN - 1)],
                    v, mask=m)
                return 0
            lax.fori_loop(0, SUB // 16, scat, 0, unroll=8)

        nsub = (b1 - s0 + SUB - 1) // SUB

        def pair(jj, _):
            j0 = 2 * jj
            issue(sub_start(j0 + 1), iB, vB, semB)
            drain(iA, vA, semA)
            scatter(j0, iA, vA)
            issue(sub_start(j0 + 2), iA, vA, semA)
            drain(iB, vB, semB)
            scatter(j0 + 1, iB, vB)
            return 0
        lax.fori_loop(0, (nsub + 1) // 2, pair, 0)
        drain(iA, vA, semA)

        tfi = tflat - base
        tmask = (tfi >= 0) & (tfi < HALF_W)
        plsc.store_scatter(
            wbuf,
            [lax.shift_right_logical(tfi, LOG_IN), tfi & (N_IN - 1)],
            tv_v[...], mask=tmask)

        pltpu.sync_copy(wbuf, w_hbm.at[pl.ds(blk * HALF_ROWS, HALF_ROWS), :])


def _densify(coarse, flat, vals, tflat, tval):
    mesh = plsc.VectorSubcoreMesh(core_axis_name="c", subcore_axis_name="s")
    nnz = flat.shape[0]
    body = lambda *refs: _sc_densify_body(nnz, *refs)
    return pl.kernel(
        body,
        out_type=jax.ShapeDtypeStruct((N_OUT, N_IN), jnp.float32),
        mesh=mesh,
        scratch_types=[
            pltpu.VMEM((128,), jnp.int32),
            pltpu.VMEM((3 * STRIDE,), jnp.int32),
            pltpu.VMEM((SUB,), jnp.int32),
            pltpu.VMEM((SUB,), jnp.float32),
            pltpu.VMEM((SUB,), jnp.int32),
            pltpu.VMEM((SUB,), jnp.float32),
            pltpu.VMEM((16,), jnp.int32),
            pltpu.VMEM((16,), jnp.float32),
            pltpu.VMEM((HALF_ROWS, N_IN), jnp.float32),
            pltpu.SemaphoreType.DMA,
            pltpu.SemaphoreType.DMA,
        ],
        compiler_params=pltpu.CompilerParams(needs_layout_passes=False),
    )(coarse, flat, vals, tflat, tval)


BM = 512


def _mm_body(x_ref, w_ref, b_ref, o_ref):
    xb = x_ref[...].astype(jnp.bfloat16)
    wb = w_ref[...].astype(jnp.bfloat16)
    acc = lax.dot_general(xb, wb, (((1,), (1,)), ((), ())),
                          preferred_element_type=jnp.float32)
    o_ref[...] = acc + b_ref[...]


def _matmul(x, w, bias2):
    batch = x.shape[0]
    return pl.pallas_call(
        _mm_body,
        grid=(batch // BM,),
        in_specs=[
            pl.BlockSpec((BM, N_IN), lambda i: (i, 0)),
            pl.BlockSpec((N_OUT, N_IN), lambda i: (0, 0)),
            pl.BlockSpec((1, N_OUT), lambda i: (0, 0)),
        ],
        out_specs=pl.BlockSpec((BM, N_OUT), lambda i: (i, 0)),
        out_shape=jax.ShapeDtypeStruct((batch, N_OUT), jnp.float32),
    )(x, w, bias2)


def kernel(in_values, values, indices, bias):
    ind = indices.astype(jnp.int32)
    nnz = ind.shape[1]
    flat = ind[0] * N_IN + ind[1]
    q = jnp.arange(0, N_OUT + 1, HALF_ROWS, dtype=jnp.int32) * N_IN
    f_sub = flat[::STRIDE]
    jc = jnp.sum(f_sub[None, :] < q[:, None], axis=1).astype(jnp.int32)
    coarse = jnp.pad(jc, (0, 128 - jc.shape[0]))
    tflat = flat[nnz - 16:]
    tval = values[nnz - 16:]
    w = _densify(coarse, flat, values, tflat, tval)
    return _matmul(in_values, w, bias.reshape(1, N_OUT))
